# baseline (speedup 1.0000x reference)
import jax
import jax.numpy as jnp
from jax import lax
from jax.experimental import pallas as pl
from jax.experimental.pallas import tpu as pltpu

EPS = 1e-8
TOPK = 12
SB = 64


def _fold8(t):
    s = t.shape[-2]
    acc = t[0:8, :]
    for i in range(8, s, 8):
        acc = acc + t[i:i + 8, :]
    return acc


def _red(t):
    return jnp.sum(jnp.sum(t, axis=2, keepdims=True), axis=1,
                   keepdims=True)[None]


def _topw(p, k):
    hh, ww = p.shape
    flat = (lax.broadcasted_iota(jnp.int32, (hh, ww), 0) * ww
            + lax.broadcasted_iota(jnp.int32, (hh, ww), 1))
    x = p
    w = jnp.zeros_like(p)
    for _ in range(k):
        m = jnp.max(x)
        fi = jnp.min(jnp.where(x == m, flat, hh * ww))
        hit = flat == fi
        w = w + hit.astype(jnp.float32)
        x = jnp.where(hit, -jnp.inf, x)
    return w


def _p12_body(th_ref, w_ref, q_ref, a_ref, d_ref, mm_ref, fgp_ref, bgp_ref,
              pf_scr, iq_scr, fs_scr, ts_scr, an_scr, ax_scr, dn_scr, dx_scr):
    b = pl.program_id(0)
    C, H, W = q_ref.shape[1], q_ref.shape[2], q_ref.shape[3]
    hw = H * W
    wv = w_ref[0]
    tf = th_ref[0]

    for hb in range(0, H, SB):
        xs = q_ref[0, :, hb:hb + SB, :]
        s2 = jnp.sum(xs * xs, axis=0)
        dw = jnp.sum(xs * wv, axis=0)
        iq = 1.0 / jnp.maximum(jnp.sqrt(s2), EPS)
        pf_scr[hb:hb + SB, :] = 1.0 / (1.0 + jnp.exp(-(dw * iq)))
        iq_scr[hb:hb + SB, :] = iq

    pf = pf_scr[...]
    mff = (pf > tf).astype(jnp.float32)
    cf = jnp.sum(_fold8(mff))
    cb = hw - cf

    for c in range(C):
        xc = q_ref[0, c]
        fs_scr[c] = _fold8(xc * mff)
        ts_scr[c] = _fold8(xc)
    fsum = _red(fs_scr[...])
    tsum = _red(ts_scr[...])

    fgp = lax.cond(cf > 0,
                   lambda: fsum / cf,
                   lambda: _red(q_ref[0] * _topw(pf, TOPK)[None]) / TOPK)
    bgp = lax.cond(cb > 0,
                   lambda: (tsum - fsum) / cb,
                   lambda: _red(q_ref[0] * _topw(1.0 - pf, TOPK)[None])
                   / TOPK)
    fgp_ref[...] = fgp
    bgp_ref[...] = bgp

    fgn = fgp[0] / jnp.maximum(jnp.sqrt(jnp.sum(fgp * fgp)), EPS)
    bgn = bgp[0] / jnp.maximum(jnp.sqrt(jnp.sum(bgp * bgp)), EPS)

    an = ax = dn = dx = None
    for hb in range(0, H, SB):
        xs = q_ref[0, :, hb:hb + SB, :]
        iq = iq_scr[hb:hb + SB, :]
        a = jnp.sum(xs * fgn, axis=0) * iq
        d = jnp.sum(xs * bgn, axis=0) * iq
        a_ref[0, hb:hb + SB, :] = a
        d_ref[0, hb:hb + SB, :] = d
        na = a[0:8, :]
        nd = d[0:8, :]
        ma = a[0:8, :]
        md = d[0:8, :]
        for i in range(8, SB, 8):
            na = jnp.minimum(na, a[i:i + 8, :])
            nd = jnp.minimum(nd, d[i:i + 8, :])
            ma = jnp.maximum(ma, a[i:i + 8, :])
            md = jnp.maximum(md, d[i:i + 8, :])
        an = na if an is None else jnp.minimum(an, na)
        dn = nd if dn is None else jnp.minimum(dn, nd)
        ax = ma if ax is None else jnp.maximum(ax, ma)
        dx = md if dx is None else jnp.maximum(dx, md)

    @pl.when(b == 0)
    def _():
        an_scr[...] = an
        ax_scr[...] = ax
        dn_scr[...] = dn
        dx_scr[...] = dx

    @pl.when(b != 0)
    def _():
        an_scr[...] = jnp.minimum(an_scr[...], an)
        ax_scr[...] = jnp.maximum(ax_scr[...], ax)
        dn_scr[...] = jnp.minimum(dn_scr[...], dn)
        dx_scr[...] = jnp.maximum(dx_scr[...], dx)

    @pl.when(b == pl.num_programs(0) - 1)
    def _():
        lane = lax.broadcasted_iota(jnp.int32, (1, 8), 1)
        row = jnp.where(lane == 0, jnp.min(an_scr[...]),
                        jnp.where(lane == 1, jnp.max(ax_scr[...]),
                                  jnp.where(lane == 2, jnp.min(dn_scr[...]),
                                            jnp.where(lane == 3,
                                                      jnp.max(dx_scr[...]),
                                                      0.0))))
        mm_ref[...] = row


def _p3_body(mm_ref, q_ref, a_ref, d_ref, o_ref):
    x = q_ref[0]
    a = a_ref[0]
    d = d_ref[0]
    an = (a - mm_ref[0]) / (mm_ref[1] - mm_ref[0])
    dn = (d - mm_ref[2]) / (mm_ref[3] - mm_ref[2])
    o_ref[0] = x * (an + (1.0 - dn))[None]


def kernel(supp_fp, supp_bp, query_fea, tau):
    bs, C, H, W = query_fea.shape
    f32 = jnp.float32
    ft = jax.nn.sigmoid(tau)
    th = jnp.stack([ft, 1.0 - ft]).astype(f32)

    fp = supp_fp[:, :, 0, 0]
    bp = supp_bp[:, :, 0, 0]
    nf = jnp.maximum(jnp.sqrt(jnp.sum(fp * fp, axis=1)), EPS)[:, None]
    nb = jnp.maximum(jnp.sqrt(jnp.sum(bp * bp, axis=1)), EPS)[:, None]
    wvec = (10.0 * (fp / nf - bp / nb)).reshape(bs, C, 1, 1)

    a, d, mm, fgp, bgp = pl.pallas_call(
        _p12_body,
        grid=(bs,),
        in_specs=[
            pl.BlockSpec(memory_space=pltpu.SMEM),
            pl.BlockSpec((1, C, 1, 1), lambda b: (b, 0, 0, 0)),
            pl.BlockSpec((1, C, H, W), lambda b: (b, 0, 0, 0)),
        ],
        out_specs=[
            pl.BlockSpec((1, H, W), lambda b: (b, 0, 0)),
            pl.BlockSpec((1, H, W), lambda b: (b, 0, 0)),
            pl.BlockSpec((1, 8), lambda b: (0, 0)),
            pl.BlockSpec((1, C, 1, 1), lambda b: (b, 0, 0, 0)),
            pl.BlockSpec((1, C, 1, 1), lambda b: (b, 0, 0, 0)),
        ],
        out_shape=[
            jax.ShapeDtypeStruct((bs, H, W), f32),
            jax.ShapeDtypeStruct((bs, H, W), f32),
            jax.ShapeDtypeStruct((1, 8), f32),
            jax.ShapeDtypeStruct((bs, C, 1, 1), f32),
            jax.ShapeDtypeStruct((bs, C, 1, 1), f32),
        ],
        scratch_shapes=[
            pltpu.VMEM((H, W), f32),
            pltpu.VMEM((H, W), f32),
            pltpu.VMEM((C, 8, W), f32),
            pltpu.VMEM((C, 8, W), f32),
            pltpu.VMEM((8, W), f32),
            pltpu.VMEM((8, W), f32),
            pltpu.VMEM((8, W), f32),
            pltpu.VMEM((8, W), f32),
        ],
    )(th, wvec, query_fea)

    qo = pl.pallas_call(
        _p3_body,
        grid=(bs,),
        in_specs=[
            pl.BlockSpec(memory_space=pltpu.SMEM),
            pl.BlockSpec((1, C, H, W), lambda b: (b, 0, 0, 0)),
            pl.BlockSpec((1, H, W), lambda b: (b, 0, 0)),
            pl.BlockSpec((1, H, W), lambda b: (b, 0, 0)),
        ],
        out_specs=pl.BlockSpec((1, C, H, W), lambda b: (b, 0, 0, 0)),
        out_shape=jax.ShapeDtypeStruct((bs, C, H, W), f32),
    )(mm.reshape(8), query_fea, a, d)

    return (qo, fgp, bgp)


# speedup vs baseline: 1.0652x; 1.0652x over previous
import jax
import jax.numpy as jnp
from jax import lax
from jax.experimental import pallas as pl
from jax.experimental.pallas import tpu as pltpu

EPS = 1e-8
TOPK = 12
SB = 32


def _fold8(t):
    s = t.shape[-2]
    acc = t[0:8, :]
    for i in range(8, s, 8):
        acc = acc + t[i:i + 8, :]
    return acc


def _red(t):
    return jnp.sum(jnp.sum(t, axis=2, keepdims=True), axis=1,
                   keepdims=True)[None]


def _topw(p, k):
    hh, ww = p.shape
    flat = (lax.broadcasted_iota(jnp.int32, (hh, ww), 0) * ww
            + lax.broadcasted_iota(jnp.int32, (hh, ww), 1))
    x = p
    w = jnp.zeros_like(p)
    for _ in range(k):
        m = jnp.max(x)
        fi = jnp.min(jnp.where(x == m, flat, hh * ww))
        hit = flat == fi
        w = w + hit.astype(jnp.float32)
        x = jnp.where(hit, -jnp.inf, x)
    return w


def _p12_body(th_ref, w_ref, q_ref, a_ref, d_ref, mm_ref, fgp_ref, bgp_ref,
              pf_scr, iq_scr, fs_scr, ts_scr, an_scr, ax_scr, dn_scr, dx_scr):
    b = pl.program_id(0)
    C, H, W = q_ref.shape[1], q_ref.shape[2], q_ref.shape[3]
    hw = H * W
    wv = w_ref[0]
    tf = th_ref[0]

    for hb in range(0, H, SB):
        xs = q_ref[0, :, hb:hb + SB, :]
        s2 = jnp.sum(xs * xs, axis=0)
        dw = jnp.sum(xs * wv, axis=0)
        iq = 1.0 / jnp.maximum(jnp.sqrt(s2), EPS)
        pf_scr[hb:hb + SB, :] = 1.0 / (1.0 + jnp.exp(-(dw * iq)))
        iq_scr[hb:hb + SB, :] = iq

    pf = pf_scr[...]
    mff = (pf > tf).astype(jnp.float32)
    cf = jnp.sum(_fold8(mff))
    cb = hw - cf

    for c in range(C):
        xc = q_ref[0, c]
        fs_scr[c] = _fold8(xc * mff)
        ts_scr[c] = _fold8(xc)
    fsum = _red(fs_scr[...])
    tsum = _red(ts_scr[...])

    fgp = lax.cond(cf > 0,
                   lambda: fsum / cf,
                   lambda: _red(q_ref[0] * _topw(pf, TOPK)[None]) / TOPK)
    bgp = lax.cond(cb > 0,
                   lambda: (tsum - fsum) / cb,
                   lambda: _red(q_ref[0] * _topw(1.0 - pf, TOPK)[None])
                   / TOPK)
    fgp_ref[...] = fgp
    bgp_ref[...] = bgp

    fgn = fgp[0] / jnp.maximum(jnp.sqrt(jnp.sum(fgp * fgp)), EPS)
    bgn = bgp[0] / jnp.maximum(jnp.sqrt(jnp.sum(bgp * bgp)), EPS)

    an = ax = dn = dx = None
    for hb in range(0, H, SB):
        xs = q_ref[0, :, hb:hb + SB, :]
        iq = iq_scr[hb:hb + SB, :]
        a = jnp.sum(xs * fgn, axis=0) * iq
        d = jnp.sum(xs * bgn, axis=0) * iq
        a_ref[0, hb:hb + SB, :] = a
        d_ref[0, hb:hb + SB, :] = d
        na = a[0:8, :]
        nd = d[0:8, :]
        ma = a[0:8, :]
        md = d[0:8, :]
        for i in range(8, SB, 8):
            na = jnp.minimum(na, a[i:i + 8, :])
            nd = jnp.minimum(nd, d[i:i + 8, :])
            ma = jnp.maximum(ma, a[i:i + 8, :])
            md = jnp.maximum(md, d[i:i + 8, :])
        an = na if an is None else jnp.minimum(an, na)
        dn = nd if dn is None else jnp.minimum(dn, nd)
        ax = ma if ax is None else jnp.maximum(ax, ma)
        dx = md if dx is None else jnp.maximum(dx, md)

    @pl.when(b == 0)
    def _():
        an_scr[...] = an
        ax_scr[...] = ax
        dn_scr[...] = dn
        dx_scr[...] = dx

    @pl.when(b != 0)
    def _():
        an_scr[...] = jnp.minimum(an_scr[...], an)
        ax_scr[...] = jnp.maximum(ax_scr[...], ax)
        dn_scr[...] = jnp.minimum(dn_scr[...], dn)
        dx_scr[...] = jnp.maximum(dx_scr[...], dx)

    @pl.when(b == pl.num_programs(0) - 1)
    def _():
        lane = lax.broadcasted_iota(jnp.int32, (1, 8), 1)
        row = jnp.where(lane == 0, jnp.min(an_scr[...]),
                        jnp.where(lane == 1, jnp.max(ax_scr[...]),
                                  jnp.where(lane == 2, jnp.min(dn_scr[...]),
                                            jnp.where(lane == 3,
                                                      jnp.max(dx_scr[...]),
                                                      0.0))))
        mm_ref[...] = row


def _p3_body(mm_ref, q_ref, a_ref, d_ref, o_ref):
    x = q_ref[0]
    a = a_ref[0]
    d = d_ref[0]
    an = (a - mm_ref[0]) / (mm_ref[1] - mm_ref[0])
    dn = (d - mm_ref[2]) / (mm_ref[3] - mm_ref[2])
    o_ref[0] = x * (an + (1.0 - dn))[None]


def kernel(supp_fp, supp_bp, query_fea, tau):
    bs, C, H, W = query_fea.shape
    f32 = jnp.float32
    ft = jax.nn.sigmoid(tau)
    th = jnp.stack([ft, 1.0 - ft]).astype(f32)

    fp = supp_fp[:, :, 0, 0]
    bp = supp_bp[:, :, 0, 0]
    nf = jnp.maximum(jnp.sqrt(jnp.sum(fp * fp, axis=1)), EPS)[:, None]
    nb = jnp.maximum(jnp.sqrt(jnp.sum(bp * bp, axis=1)), EPS)[:, None]
    wvec = (10.0 * (fp / nf - bp / nb)).reshape(bs, C, 1, 1)

    a, d, mm, fgp, bgp = pl.pallas_call(
        _p12_body,
        grid=(bs,),
        in_specs=[
            pl.BlockSpec(memory_space=pltpu.SMEM),
            pl.BlockSpec((1, C, 1, 1), lambda b: (b, 0, 0, 0)),
            pl.BlockSpec((1, C, H, W), lambda b: (b, 0, 0, 0)),
        ],
        out_specs=[
            pl.BlockSpec((1, H, W), lambda b: (b, 0, 0)),
            pl.BlockSpec((1, H, W), lambda b: (b, 0, 0)),
            pl.BlockSpec((1, 8), lambda b: (0, 0)),
            pl.BlockSpec((1, C, 1, 1), lambda b: (b, 0, 0, 0)),
            pl.BlockSpec((1, C, 1, 1), lambda b: (b, 0, 0, 0)),
        ],
        out_shape=[
            jax.ShapeDtypeStruct((bs, H, W), f32),
            jax.ShapeDtypeStruct((bs, H, W), f32),
            jax.ShapeDtypeStruct((1, 8), f32),
            jax.ShapeDtypeStruct((bs, C, 1, 1), f32),
            jax.ShapeDtypeStruct((bs, C, 1, 1), f32),
        ],
        scratch_shapes=[
            pltpu.VMEM((H, W), f32),
            pltpu.VMEM((H, W), f32),
            pltpu.VMEM((C, 8, W), f32),
            pltpu.VMEM((C, 8, W), f32),
            pltpu.VMEM((8, W), f32),
            pltpu.VMEM((8, W), f32),
            pltpu.VMEM((8, W), f32),
            pltpu.VMEM((8, W), f32),
        ],
    )(th, wvec, query_fea)

    qo = pl.pallas_call(
        _p3_body,
        grid=(bs,),
        in_specs=[
            pl.BlockSpec(memory_space=pltpu.SMEM),
            pl.BlockSpec((1, C, H, W), lambda b: (b, 0, 0, 0)),
            pl.BlockSpec((1, H, W), lambda b: (b, 0, 0)),
            pl.BlockSpec((1, H, W), lambda b: (b, 0, 0)),
        ],
        out_specs=pl.BlockSpec((1, C, H, W), lambda b: (b, 0, 0, 0)),
        out_shape=jax.ShapeDtypeStruct((bs, C, H, W), f32),
    )(mm.reshape(8), query_fea, a, d)

    return (qo, fgp, bgp)
